# baseline (speedup 1.0000x reference)
import functools

import jax
import jax.numpy as jnp
from jax import lax
from jax.experimental import pallas as pl
from jax.experimental.pallas import tpu as pltpu
from jax.experimental.pallas import tpu_sc as plsc

BATCH = 4096
HIST = 50
D = 32
NC, NS = 2, 16
NW = NC * NS
B = BATCH * HIST
B_PER_W = B // NW
CHUNK = 128
G = B_PER_W // CHUNK


def _make_kernel():
    mesh = plsc.VectorSubcoreMesh(core_axis_name="c", subcore_axis_name="s")

    @functools.partial(
        pl.kernel,
        mesh=mesh,
        out_type=jax.ShapeDtypeStruct((B, D), jnp.float32),
        scratch_types=[
            pltpu.VMEM((G, CHUNK), jnp.int32),
            pltpu.VMEM((CHUNK, D), jnp.float32),
            pltpu.SemaphoreType.DMA,
        ],
        compiler_params=pltpu.CompilerParams(use_tc_tiling_on_sc=False),
    )
    def k(idx_hbm, table_hbm, out_hbm, idx_v, rows_v, sem):
        wid = lax.axis_index("s") * NC + lax.axis_index("c")
        base = wid * B_PER_W
        pltpu.sync_copy(idx_hbm.at[wid], idx_v)

        def body(j, carry):
            pltpu.async_copy(table_hbm.at[idx_v.at[j]], rows_v, sem).wait()
            pltpu.sync_copy(rows_v, out_hbm.at[pl.ds(base + j * CHUNK, CHUNK)])
            return carry

        lax.fori_loop(0, G, body, 0)

    return k


_gather_kernel = _make_kernel()


def kernel(inputs, embeddings):
    idx = inputs.astype(jnp.int32).reshape(NW, G, CHUNK)
    out = _gather_kernel(idx, embeddings)
    return out.reshape(BATCH, HIST, D)


# speedup vs baseline: 1.1401x; 1.1401x over previous
import functools

import jax
import jax.numpy as jnp
from jax import lax
from jax.experimental import pallas as pl
from jax.experimental.pallas import tpu as pltpu
from jax.experimental.pallas import tpu_sc as plsc

BATCH = 4096
HIST = 50
D = 32
VOCAB = 1000000
NC, NS = 2, 16
NW = NC * NS
CTILE = 128
NCT = BATCH // CTILE


def _make_kernel():
    mesh = plsc.VectorSubcoreMesh(core_axis_name="c", subcore_axis_name="s")

    @functools.partial(
        pl.kernel,
        mesh=mesh,
        out_type=jax.ShapeDtypeStruct((HIST * 4 * NCT, 8 * CTILE), jnp.float32),
        scratch_types=[
            pltpu.VMEM((HIST, CTILE), jnp.int32),
            pltpu.VMEM((CTILE, D), jnp.float32),
            pltpu.VMEM((4 * 8 * CTILE,), jnp.float32),
            pltpu.SemaphoreType.DMA,
        ],
        compiler_params=pltpu.CompilerParams(
            use_tc_tiling_on_sc=False, needs_layout_passes=False
        ),
    )
    def k(idx_hbm, table_hbm, out_hbm, idx_v, rows_v, out_v, sem):
        wid = lax.axis_index("s") * NC + lax.axis_index("c")
        pltpu.sync_copy(idx_hbm.at[:, wid], idx_v)

        def body(h, carry):
            lanes = lax.iota(jnp.int32, 16)
            pltpu.async_copy(table_hbm.at[idx_v.at[h]], rows_v, sem).wait()
            for r in range(4):
                for dlo in range(8):
                    d = 8 * r + dlo
                    dvec = jnp.full((16,), d, jnp.int32)
                    for kk in range(8):
                        blo = lanes + (16 * kk)
                        v = plsc.load_gather(rows_v, [blo, dvec])
                        out_v[pl.ds((r * 8 + dlo) * CTILE + 16 * kk, 16)] = v
            for r in range(4):
                pltpu.sync_copy(out_v.at[pl.ds(r * 8 * CTILE, 8 * CTILE)], out_hbm.at[(h * 4 + r) * NCT + wid])
            return carry

        lax.fori_loop(0, HIST, body, 0)

    return k


_gather_kernel = _make_kernel()


def kernel(inputs, embeddings):
    idx3 = inputs.T.astype(jnp.int32).reshape(HIST * BATCH).reshape(HIST, NCT, CTILE)
    emb_fat = jax.lax.optimization_barrier(embeddings.reshape(VOCAB // 4, 4 * D))
    emb2 = emb_fat.reshape(VOCAB, D)
    out = _gather_kernel(idx3, emb2)
    out5 = out.reshape(HIST, 4, NCT, 8, CTILE)
    return out5.transpose(2, 4, 0, 1, 3).reshape(BATCH, HIST, D)


# speedup vs baseline: 1.2051x; 1.0570x over previous
import functools

import jax
import jax.numpy as jnp
from jax import lax
from jax.experimental import pallas as pl
from jax.experimental.pallas import tpu as pltpu
from jax.experimental.pallas import tpu_sc as plsc

BATCH = 4096
HIST = 50
D = 32
VOCAB = 1000000
NC, NS = 2, 16
NW = NC * NS
CTILE = 128
NCT = BATCH // CTILE


def _make_kernel():
    mesh = plsc.VectorSubcoreMesh(core_axis_name="c", subcore_axis_name="s")

    @functools.partial(
        pl.kernel,
        mesh=mesh,
        out_type=jax.ShapeDtypeStruct((HIST, 4, NCT, 8, CTILE), jnp.float32),
        scratch_types=[
            pltpu.VMEM((HIST, CTILE), jnp.int32),
            pltpu.VMEM((2, CTILE, D), jnp.float32),
            pltpu.VMEM((2, 4, 8, CTILE), jnp.float32),
            pltpu.SemaphoreType.DMA,
            pltpu.SemaphoreType.DMA,
            pltpu.SemaphoreType.DMA,
            pltpu.SemaphoreType.DMA,
        ],
        compiler_params=pltpu.CompilerParams(
            use_tc_tiling_on_sc=False, needs_layout_passes=False
        ),
    )
    def k(idx_hbm, table_hbm, out_hbm, idx_v, rows_v, out_v, g0, g1, o0, o1):
        wid = lax.axis_index("s") * NC + lax.axis_index("c")
        gsem = (g0, g1)
        osem = (o0, o1)
        pltpu.sync_copy(idx_hbm.at[:, wid], idx_v)

        def gather_start(h, slot):
            pltpu.async_copy(table_hbm.at[idx_v.at[h]], rows_v.at[slot], gsem[slot])

        def gather_wait(slot):
            pltpu.make_async_copy(
                table_hbm.at[idx_v.at[0]], rows_v.at[slot], gsem[slot]
            ).wait()

        def out_start(h, slot):
            pltpu.async_copy(out_v.at[slot], out_hbm.at[h, :, wid], osem[slot])

        def out_wait(h, slot):
            pltpu.make_async_copy(
                out_v.at[slot], out_hbm.at[h, :, wid], osem[slot]
            ).wait()

        def transpose_block(slot):
            lanes = lax.iota(jnp.int32, 16)
            rv = rows_v.at[slot]
            for r in range(4):
                for dlo in range(8):
                    d = 8 * r + dlo
                    dvec = jnp.full((16,), d, jnp.int32)
                    for kk in range(8):
                        blo = lanes + (16 * kk)
                        v = plsc.load_gather(rv, [blo, dvec])
                        out_v[slot, r, dlo, pl.ds(16 * kk, 16)] = v

        def step(h, slot, nslot):
            @pl.when(h + 1 < HIST)
            def _():
                gather_start(h + 1, nslot)

            gather_wait(slot)

            @pl.when(h >= 2)
            def _():
                out_wait(h - 2, slot)

            transpose_block(slot)
            out_start(h, slot)

        gather_start(0, 0)

        def body(i, carry):
            step(2 * i, 0, 1)
            step(2 * i + 1, 1, 0)
            return carry

        lax.fori_loop(0, HIST // 2, body, 0)
        out_wait(HIST - 2, 0)
        out_wait(HIST - 1, 1)

    return k


_gather_kernel = _make_kernel()


def kernel(inputs, embeddings):
    idx3 = inputs.T.astype(jnp.int32).reshape(HIST * BATCH).reshape(HIST, NCT, CTILE)
    emb_fat = jax.lax.optimization_barrier(embeddings.reshape(VOCAB // 4, 4 * D))
    emb2 = emb_fat.reshape(VOCAB, D)
    out5 = _gather_kernel(idx3, emb2)
    return out5.transpose(2, 4, 0, 1, 3).reshape(BATCH, HIST, D)
